# baseline (speedup 1.0000x reference)
import functools

import jax
import jax.numpy as jnp
import numpy as np
from jax.experimental import pallas as pl
from jax.experimental.pallas import tpu as pltpu

_TOP_K = 8
_N_EXPERT = 64
_BLOCK_T = 256


def _threefry2x32(k0, k1, x0, x1):
    rot_a = (13, 15, 26, 6)
    rot_b = (17, 29, 16, 24)
    ks0 = np.uint32(k0)
    ks1 = np.uint32(k1)
    ks2 = np.uint32(ks0 ^ ks1 ^ np.uint32(0x1BD11BDA))

    def rotl(v, d):
        return (v << np.uint32(d)) | (v >> np.uint32(32 - d))

    x0 = x0 + ks0
    x1 = x1 + ks1
    schedule = [(rot_a, ks1, ks2, 1), (rot_b, ks2, ks0, 2),
                (rot_a, ks0, ks1, 3), (rot_b, ks1, ks2, 4),
                (rot_a, ks2, ks0, 5)]
    for rots, add0, add1, inc in schedule:
        for r in rots:
            x0 = x0 + x1
            x1 = rotl(x1, r)
            x1 = x1 ^ x0
        x0 = x0 + add0
        x1 = x1 + add1 + np.uint32(inc)
    return x0, x1


@functools.lru_cache(maxsize=None)
def _uniform_const_t(rows: int):
    n = rows * _N_EXPERT
    with np.errstate(over="ignore"):
        cnt = np.arange(n, dtype=np.uint64)
        hi = (cnt >> np.uint64(32)).astype(np.uint32)
        lo = cnt.astype(np.uint32)
        b0, b1 = _threefry2x32(0, 42, hi, lo)
    bits = b0 ^ b1
    f = ((bits >> np.uint32(9)) | np.uint32(0x3F800000)).view(np.float32)
    u = (f - np.float32(1.0)).reshape(rows, _N_EXPERT)
    return np.ascontiguousarray(u.T)


def _router_block(x_ref, wc_ref, u_ref, bias_ref, probs_ref, idx_ref):
    xb = x_ref[...]
    wc2 = wc_ref[...]
    acc = jax.lax.dot_general(
        wc2, xb, (((1,), (1,)), ((), ())),
        preferred_element_type=jnp.float32,
        precision=jax.lax.Precision.DEFAULT)
    logits = acc[:_N_EXPERT, :]
    noise_logits = acc[_N_EXPERT:, :]
    noisy = (u_ref[...] * jax.nn.softplus(noise_logits)
             + logits + bias_ref[...])
    t = noisy.shape[1]
    iota_f = jax.lax.broadcasted_iota(
        jnp.int32, (_N_EXPERT, t), 0).astype(jnp.float32)
    work = noisy
    selected = jnp.zeros((_N_EXPERT, t), jnp.bool_)
    gmax = None
    picks = []
    for j in range(_TOP_K):
        m = jnp.max(work, axis=0, keepdims=True)
        if j == 0:
            gmax = m
        cand = jnp.where(work == m, iota_f, float(_N_EXPERT))
        pick = jnp.min(cand, axis=0, keepdims=True)
        sel = cand == pick
        picks.append(pick)
        work = jnp.where(sel, -jnp.inf, work)
        selected = jnp.logical_or(selected, sel)
    idx_ref[...] = jnp.concatenate(picks, axis=0).astype(jnp.int32)
    e = jnp.where(selected, jnp.exp(noisy - gmax), 0.0)
    z = jnp.sum(e, axis=0, keepdims=True)
    probs_ref[...] = e / z


def kernel(x, W_router, W_noise, bias):
    b, s, d = x.shape
    rows = b * s
    xf = x.reshape(rows, d)
    wc2 = jnp.concatenate([W_router, W_noise], axis=0)
    ut = jnp.asarray(_uniform_const_t(rows))
    bias2 = bias.reshape(_N_EXPERT, 1)

    grid = (rows // _BLOCK_T,)
    probs_t, idx_t = pl.pallas_call(
        _router_block,
        grid=grid,
        in_specs=[
            pl.BlockSpec((_BLOCK_T, d), lambda i: (i, 0)),
            pl.BlockSpec((2 * _N_EXPERT, d), lambda i: (0, 0)),
            pl.BlockSpec((_N_EXPERT, _BLOCK_T), lambda i: (0, i)),
            pl.BlockSpec((_N_EXPERT, 1), lambda i: (0, 0)),
        ],
        out_specs=[
            pl.BlockSpec((_N_EXPERT, _BLOCK_T), lambda i: (0, i)),
            pl.BlockSpec((_TOP_K, _BLOCK_T), lambda i: (0, i)),
        ],
        out_shape=[
            jax.ShapeDtypeStruct((_N_EXPERT, rows), jnp.float32),
            jax.ShapeDtypeStruct((_TOP_K, rows), jnp.int32),
        ],
    )(xf, wc2, ut, bias2)
    probs = probs_t.T.reshape(b, s, _N_EXPERT)
    idx = idx_t.T.reshape(b, s, _TOP_K)
    return probs, idx


# speedup vs baseline: 1.2386x; 1.2386x over previous
import functools

import jax
import jax.numpy as jnp
import numpy as np
from jax.experimental import pallas as pl
from jax.experimental.pallas import tpu as pltpu

_TOP_K = 8
_N_EXPERT = 64
_BLOCK_T = 1024


def _threefry2x32(k0, k1, x0, x1):
    rot_a = (13, 15, 26, 6)
    rot_b = (17, 29, 16, 24)
    ks0 = np.uint32(k0)
    ks1 = np.uint32(k1)
    ks2 = np.uint32(ks0 ^ ks1 ^ np.uint32(0x1BD11BDA))

    def rotl(v, d):
        return (v << np.uint32(d)) | (v >> np.uint32(32 - d))

    x0 = x0 + ks0
    x1 = x1 + ks1
    schedule = [(rot_a, ks1, ks2, 1), (rot_b, ks2, ks0, 2),
                (rot_a, ks0, ks1, 3), (rot_b, ks1, ks2, 4),
                (rot_a, ks2, ks0, 5)]
    for rots, add0, add1, inc in schedule:
        for r in rots:
            x0 = x0 + x1
            x1 = rotl(x1, r)
            x1 = x1 ^ x0
        x0 = x0 + add0
        x1 = x1 + add1 + np.uint32(inc)
    return x0, x1


@functools.lru_cache(maxsize=None)
def _uniform_const_t(rows: int):
    n = rows * _N_EXPERT
    with np.errstate(over="ignore"):
        cnt = np.arange(n, dtype=np.uint64)
        hi = (cnt >> np.uint64(32)).astype(np.uint32)
        lo = cnt.astype(np.uint32)
        b0, b1 = _threefry2x32(0, 42, hi, lo)
    bits = b0 ^ b1
    f = ((bits >> np.uint32(9)) | np.uint32(0x3F800000)).view(np.float32)
    u = (f - np.float32(1.0)).reshape(rows, _N_EXPERT)
    return np.ascontiguousarray(u.T)


def _router_block(x_ref, wc_ref, u_ref, bias_ref, probs_ref, idx_ref):
    xb = x_ref[...]
    wc2 = wc_ref[...]
    acc = jax.lax.dot_general(
        wc2, xb, (((1,), (1,)), ((), ())),
        preferred_element_type=jnp.float32,
        precision=jax.lax.Precision.DEFAULT)
    logits = acc[:_N_EXPERT, :]
    noise_logits = acc[_N_EXPERT:, :]
    noisy = (u_ref[...] * jax.nn.softplus(noise_logits)
             + logits + bias_ref[...])
    t = noisy.shape[1]
    iota_f = jax.lax.broadcasted_iota(
        jnp.int32, (_N_EXPERT, t), 0).astype(jnp.float32)
    work = noisy
    selected = jnp.zeros((_N_EXPERT, t), jnp.bool_)
    gmax = None
    picks = []
    for j in range(_TOP_K):
        m = jnp.max(work, axis=0, keepdims=True)
        if j == 0:
            gmax = m
        cand = jnp.where(work == m, iota_f, float(_N_EXPERT))
        pick = jnp.min(cand, axis=0, keepdims=True)
        sel = cand == pick
        picks.append(pick)
        work = jnp.where(sel, -jnp.inf, work)
        selected = jnp.logical_or(selected, sel)
    idx_ref[...] = jnp.concatenate(picks, axis=0).astype(jnp.int32)
    e = jnp.where(selected, jnp.exp(noisy - gmax), 0.0)
    z = jnp.sum(e, axis=0, keepdims=True)
    probs_ref[...] = e / z


def kernel(x, W_router, W_noise, bias):
    b, s, d = x.shape
    rows = b * s
    xf = x.reshape(rows, d)
    wc2 = jnp.concatenate([W_router, W_noise], axis=0)
    ut = jnp.asarray(_uniform_const_t(rows))
    bias2 = bias.reshape(_N_EXPERT, 1)

    grid = (rows // _BLOCK_T,)
    probs_t, idx_t = pl.pallas_call(
        _router_block,
        grid=grid,
        in_specs=[
            pl.BlockSpec((_BLOCK_T, d), lambda i: (i, 0)),
            pl.BlockSpec((2 * _N_EXPERT, d), lambda i: (0, 0)),
            pl.BlockSpec((_N_EXPERT, _BLOCK_T), lambda i: (0, i)),
            pl.BlockSpec((_N_EXPERT, 1), lambda i: (0, 0)),
        ],
        out_specs=[
            pl.BlockSpec((_N_EXPERT, _BLOCK_T), lambda i: (0, i)),
            pl.BlockSpec((_TOP_K, _BLOCK_T), lambda i: (0, i)),
        ],
        out_shape=[
            jax.ShapeDtypeStruct((_N_EXPERT, rows), jnp.float32),
            jax.ShapeDtypeStruct((_TOP_K, rows), jnp.int32),
        ],
    )(xf, wc2, ut, bias2)
    probs = probs_t.T.reshape(b, s, _N_EXPERT)
    idx = idx_t.T.reshape(b, s, _TOP_K)
    return probs, idx


# speedup vs baseline: 1.2488x; 1.0082x over previous
import jax
import jax.numpy as jnp
from jax.experimental import pallas as pl
from jax.experimental.pallas import tpu as pltpu

_TOP_K = 8
_N_EXPERT = 64
_BLOCK_T = 1024

_KS0 = 0
_KS1 = 42
_KS2 = _KS0 ^ _KS1 ^ 0x1BD11BDA


def _uniform_block(block_idx, t):
    tok = jax.lax.broadcasted_iota(jnp.int32, (_N_EXPERT, t), 1)
    exp_i = jax.lax.broadcasted_iota(jnp.int32, (_N_EXPERT, t), 0)
    cnt = ((block_idx * t + tok) * _N_EXPERT + exp_i).astype(jnp.uint32)

    def rotl(v, d):
        return (v << jnp.uint32(d)) | (v >> jnp.uint32(32 - d))

    x0 = jnp.full(cnt.shape, jnp.uint32(_KS0), jnp.uint32)
    x1 = cnt + jnp.uint32(_KS1)
    rot_a = (13, 15, 26, 6)
    rot_b = (17, 29, 16, 24)
    schedule = ((rot_a, _KS1, _KS2, 1), (rot_b, _KS2, _KS0, 2),
                (rot_a, _KS0, _KS1, 3), (rot_b, _KS1, _KS2, 4),
                (rot_a, _KS2, _KS0, 5))
    for rots, add0, add1, inc in schedule:
        for r in rots:
            x0 = x0 + x1
            x1 = rotl(x1, r)
            x1 = x1 ^ x0
        x0 = x0 + jnp.uint32(add0)
        x1 = x1 + jnp.uint32((add1 + inc) & 0xFFFFFFFF)
    bits = x0 ^ x1
    f = jax.lax.bitcast_convert_type(
        (bits >> jnp.uint32(9)) | jnp.uint32(0x3F800000), jnp.float32)
    return f - jnp.float32(1.0)


def _router_block(x_ref, wc_ref, bias_ref, probs_ref, idx_ref):
    xb = x_ref[...]
    wc2 = wc_ref[...]
    acc = jax.lax.dot_general(
        wc2, xb, (((1,), (1,)), ((), ())),
        preferred_element_type=jnp.float32,
        precision=jax.lax.Precision.DEFAULT)
    logits = acc[:_N_EXPERT, :]
    noise_logits = acc[_N_EXPERT:, :]
    t = acc.shape[1]
    u = _uniform_block(pl.program_id(0), t)
    noisy = u * jax.nn.softplus(noise_logits) + logits + bias_ref[...]
    iota_f = jax.lax.broadcasted_iota(
        jnp.int32, (_N_EXPERT, t), 0).astype(jnp.float32)
    work = noisy
    selected = jnp.zeros((_N_EXPERT, t), jnp.bool_)
    gmax = None
    picks = []
    for j in range(_TOP_K):
        m = jnp.max(work, axis=0, keepdims=True)
        if j == 0:
            gmax = m
        cand = jnp.where(work == m, iota_f, float(_N_EXPERT))
        pick = jnp.min(cand, axis=0, keepdims=True)
        sel = cand == pick
        picks.append(pick)
        work = jnp.where(sel, -jnp.inf, work)
        selected = jnp.logical_or(selected, sel)
    idx_ref[...] = jnp.concatenate(picks, axis=0).astype(jnp.int32)
    e = jnp.where(selected, jnp.exp(noisy - gmax), 0.0)
    z = jnp.sum(e, axis=0, keepdims=True)
    probs_ref[...] = e / z


def kernel(x, W_router, W_noise, bias):
    b, s, d = x.shape
    rows = b * s
    xf = x.reshape(rows, d)
    wc2 = jnp.concatenate([W_router, W_noise], axis=0)
    bias2 = bias.reshape(_N_EXPERT, 1)

    grid = (rows // _BLOCK_T,)
    probs_t, idx_t = pl.pallas_call(
        _router_block,
        grid=grid,
        in_specs=[
            pl.BlockSpec((_BLOCK_T, d), lambda i: (i, 0)),
            pl.BlockSpec((2 * _N_EXPERT, d), lambda i: (0, 0)),
            pl.BlockSpec((_N_EXPERT, 1), lambda i: (0, 0)),
        ],
        out_specs=[
            pl.BlockSpec((_N_EXPERT, _BLOCK_T), lambda i: (0, i)),
            pl.BlockSpec((_TOP_K, _BLOCK_T), lambda i: (0, i)),
        ],
        out_shape=[
            jax.ShapeDtypeStruct((_N_EXPERT, rows), jnp.float32),
            jax.ShapeDtypeStruct((_TOP_K, rows), jnp.int32),
        ],
    )(xf, wc2, bias2)
    probs = probs_t.T.reshape(b, s, _N_EXPERT)
    idx = idx_t.T.reshape(b, s, _TOP_K)
    return probs, idx
